# Initial kernel scaffold; baseline (speedup 1.0000x reference)
import jax
import jax.numpy as jnp
from jax.experimental import pallas as pl


def kernel(x, w1, b1, w2, b2, w3, b3, w4, b4, w5, b5, w6, b6, fc_wt, fc_b, fc_gamma, fc_beta):
    raise NotImplementedError("write your pallas kernel here")


# speedup vs baseline: 3.2640x; 3.2640x over previous
import jax
import jax.numpy as jnp
from jax import lax
from jax.experimental import pallas as pl
from jax.experimental.pallas import tpu as pltpu

_B = 16


def _elu(a):
    return jnp.where(a > 0.0, a, jnp.exp(jnp.minimum(a, 0.0)) - 1.0)


def _conv_mm(fp, w_ref, b_ref, h, w):
    bsz = fp.shape[0]
    pats = jnp.concatenate(
        [fp[:, kh:kh + h, kw:kw + w, :] for kh in range(3) for kw in range(3)],
        axis=-1)
    k = w_ref.shape[0]
    a2 = pats.reshape(bsz * h * w, k)
    y = jnp.dot(a2, w_ref[...], preferred_element_type=jnp.float32)
    y = y + b_ref[...]
    return _elu(y).reshape(bsz, h, w, w_ref.shape[1])


def _pool2(f):
    bsz, h, w, c = f.shape
    m1 = jnp.max(f.reshape(bsz, h // 2, 2, w, c), axis=2)
    m1 = m1.reshape(bsz, h // 2, w // 2, 2, c)
    return jnp.max(m1, axis=3)


def _pad_hw(f):
    return jnp.pad(f, ((0, 0), (1, 1), (1, 1), (0, 0)))


def _net_kernel(x_ref, w1_ref, b1_ref, w2_ref, b2_ref, w3_ref, b3_ref,
                w4_ref, b4_ref, w5_ref, b5_ref, w6_ref, b6_ref,
                fcw_ref, fcb_ref, o_ref):
    bsz = x_ref.shape[0]
    x = x_ref[...]
    xp = jnp.pad(x, ((0, 0), (1, 1), (1, 1)))
    acc = jnp.zeros((bsz, 24, 24, 32), jnp.float32)
    for kh in range(3):
        for kw in range(3):
            acc = acc + xp[:, kh:kh + 24, kw:kw + 24, None] * w1_ref[kh, kw, 0]
    f = _elu(acc + b1_ref[...].reshape(1, 1, 1, 32))

    f = _conv_mm(_pad_hw(f), w2_ref, b2_ref, 24, 24)
    f = _pool2(f)
    f = _conv_mm(_pad_hw(f), w3_ref, b3_ref, 12, 12)
    f = _conv_mm(_pad_hw(f), w4_ref, b4_ref, 12, 12)
    f = _pool2(f)
    f = _conv_mm(_pad_hw(f), w5_ref, b5_ref, 6, 6)
    f = _conv_mm(_pad_hw(f), w6_ref, b6_ref, 6, 6)
    f = _pool2(f)

    z = f.reshape(bsz, 9 * 128)
    o_ref[...] = jnp.dot(z, fcw_ref[...],
                         preferred_element_type=jnp.float32) + fcb_ref[...]


def _bn_kernel(y_ref, g_ref, be_ref, o_ref):
    y = y_ref[...]
    mu = jnp.mean(y, axis=0, keepdims=True)
    var = jnp.mean((y - mu) ** 2, axis=0, keepdims=True)
    o_ref[...] = (y - mu) * lax.rsqrt(var + 1e-5) * g_ref[...] + be_ref[...]


def _full(shape):
    nd = len(shape)
    return pl.BlockSpec(shape, lambda i, _n=nd: (0,) * _n)


def kernel(x, w1, b1, w2, b2, w3, b3, w4, b4, w5, b5, w6, b6,
           fc_wt, fc_b, fc_gamma, fc_beta):
    n = x.shape[0]
    x3 = x.reshape(n, 24, 24)
    w2r = w2.reshape(-1, w2.shape[3])
    w3r = w3.reshape(-1, w3.shape[3])
    w4r = w4.reshape(-1, w4.shape[3])
    w5r = w5.reshape(-1, w5.shape[3])
    w6r = w6.reshape(-1, w6.shape[3])

    in_specs = [pl.BlockSpec((_B, 24, 24), lambda i: (i, 0, 0)),
                _full(w1.shape), _full(b1.shape),
                _full(w2r.shape), _full(b2.shape),
                _full(w3r.shape), _full(b3.shape),
                _full(w4r.shape), _full(b4.shape),
                _full(w5r.shape), _full(b5.shape),
                _full(w6r.shape), _full(b6.shape),
                _full(fc_wt.shape), _full(fc_b.shape)]

    y = pl.pallas_call(
        _net_kernel,
        out_shape=jax.ShapeDtypeStruct((n, 128), jnp.float32),
        grid=(n // _B,),
        in_specs=in_specs,
        out_specs=pl.BlockSpec((_B, 128), lambda i: (i, 0)),
        compiler_params=pltpu.CompilerParams(
            dimension_semantics=("parallel",)),
    )(x3, w1, b1, w2r, b2, w3r, b3, w4r, b4, w5r, b5, w6r, b6, fc_wt, fc_b)

    return pl.pallas_call(
        _bn_kernel,
        out_shape=jax.ShapeDtypeStruct((n, 128), jnp.float32),
        grid=(1,),
        in_specs=[_full((n, 128)), _full(fc_gamma.shape),
                  _full(fc_beta.shape)],
        out_specs=pl.BlockSpec((n, 128), lambda i: (0, 0)),
    )(y, fc_gamma, fc_beta)
